# Initial kernel scaffold; baseline (speedup 1.0000x reference)
import jax
import jax.numpy as jnp
from jax.experimental import pallas as pl


def kernel(x, edge_index, W, b):
    raise NotImplementedError("write your pallas kernel here")


# speedup vs baseline: 23.5336x; 23.5336x over previous
import dataclasses
import functools

import jax
import jax.numpy as jnp
from jax import lax
from jax.experimental import pallas as pl
from jax.experimental.pallas import tpu as pltpu
from jax.experimental.pallas import tpu_sc as plsc

NC = 2
NS = 16
NW = NC * NS
C = 128
DUMMY = 16


def _mesh():
    return plsc.VectorSubcoreMesh(
        core_axis_name="c", subcore_axis_name="s", num_cores=NC, num_subcores=NS
    )


def _deg_kernel(cols3, nacc, k):
    @functools.partial(
        pl.kernel,
        out_type=jax.ShapeDtypeStruct((NW * nacc,), jnp.float32),
        mesh=_mesh(),
        scratch_types=[
            pltpu.VMEM((k, C), jnp.int32),
            pltpu.VMEM((nacc,), jnp.float32),
        ],
        compiler_params=dataclasses.replace(
            pltpu.CompilerParams(), needs_layout_passes=False
        ),
    )
    def kern(cols_hbm, out_hbm, colv, hist):
        c = lax.axis_index("c")
        s = lax.axis_index("s")
        wid = s * NC + c
        pltpu.sync_copy(cols_hbm.at[wid], colv)

        @pl.loop(0, nacc // 16)
        def _(i):
            hist[pl.ds(i * 16, 16)] = jnp.zeros((16,), jnp.float32)

        ones = jnp.ones((16,), jnp.float32)

        @pl.loop(0, k)
        def _(kk):
            @pl.loop(0, C // 16)
            def _(j):
                idx = colv[kk, pl.ds(j * 16, 16)]
                plsc.addupdate_scatter(hist, [idx], ones)

        pltpu.sync_copy(hist, out_hbm.at[pl.ds(wid * nacc, nacc)])

    return kern(cols3)


def _agg_kernel(y, rows3, cols3, zeros, nacc, k, d):
    rpt = nacc // NS

    @functools.partial(
        pl.kernel,
        out_type=jax.ShapeDtypeStruct((NC, nacc, d), jnp.float32),
        mesh=_mesh(),
        scratch_types=[
            pltpu.VMEM((k, C), jnp.int32),
            pltpu.VMEM((k, C), jnp.int32),
            pltpu.VMEM((C, d), jnp.float32),
            pltpu.VMEM_SHARED((nacc, d), jnp.float32),
            pltpu.SemaphoreType.DMA,
        ],
    )
    def kern(y_hbm, rows_hbm, cols_hbm, zeros_hbm, out_hbm, rowv, colv, buf, acc, sem):
        c = lax.axis_index("c")
        s = lax.axis_index("s")
        wid = s * NC + c
        pltpu.sync_copy(zeros_hbm, acc.at[pl.ds(s * rpt, rpt)])
        pltpu.sync_copy(rows_hbm.at[wid], rowv)
        pltpu.sync_copy(cols_hbm.at[wid], colv)
        plsc.subcore_barrier()

        @pl.loop(0, k)
        def _(kk):
            pltpu.async_copy(y_hbm.at[rowv.at[kk]], buf, sem).wait()
            pltpu.sync_copy(buf, acc.at[colv.at[kk]], add=True)

        plsc.subcore_barrier()
        pltpu.sync_copy(
            acc.at[pl.ds(s * rpt, rpt)], out_hbm.at[c, pl.ds(s * rpt, rpt)]
        )

    return kern(y, rows3, cols3, zeros)


def _tc_scale0(d2, x):
    n, d = x.shape

    def body(d_ref, x_ref, y_ref):
        deg = jnp.sum(d_ref[...], axis=0) + 1.0
        dis = lax.rsqrt(deg)
        y_ref[...] = x_ref[...] * dis[:, None]

    return pl.pallas_call(
        body, out_shape=jax.ShapeDtypeStruct((n, d), jnp.float32)
    )(d2, x)


def _tc_scale1(d2, sp, y0):
    n, d = y0.shape

    def body(d_ref, s_ref, y0_ref, out_ref):
        deg = jnp.sum(d_ref[...], axis=0) + 1.0
        t = s_ref[0, pl.ds(0, n), :] + s_ref[1, pl.ds(0, n), :] + y0_ref[...]
        out_ref[...] = t * (1.0 / deg)[:, None]

    return pl.pallas_call(
        body, out_shape=jax.ShapeDtypeStruct((n, d), jnp.float32)
    )(d2, sp, y0)


def _tc_final(d2, sp, y1, W, b):
    n, d = y1.shape

    def body(d_ref, s_ref, y1_ref, w_ref, b_ref, out_ref):
        deg = jnp.sum(d_ref[...], axis=0) + 1.0
        dis = lax.rsqrt(deg)
        t = s_ref[0, pl.ds(0, n), :] + s_ref[1, pl.ds(0, n), :] + y1_ref[...]
        t = t * dis[:, None]
        out_ref[...] = (
            lax.dot_general(
                t, w_ref[...], (((1,), (1,)), ((), ())),
                preferred_element_type=jnp.float32,
            )
            + b_ref[...][None, :]
        )

    return pl.pallas_call(
        body, out_shape=jax.ShapeDtypeStruct((n, W.shape[0]), jnp.float32)
    )(d2, sp, y1, W, b)


def kernel(x, edge_index, W, b):
    n, d = x.shape
    e = edge_index.shape[1]

    k = -(-e // (NW * C))
    e_pad = NW * C * k
    pad = e_pad - e
    nacc = -(-(n + DUMMY) // (NS * 8)) * (NS * 8)
    rpt = nacc // NS

    row = edge_index[0].astype(jnp.int32)
    col = edge_index[1].astype(jnp.int32)
    ar = jnp.arange(pad, dtype=jnp.int32)
    rowp = jnp.concatenate([row, (ar * 37) % n])
    colp = jnp.concatenate([col, n + (ar % DUMMY)])
    rows3 = rowp.reshape(NW, k, C)
    cols3 = colp.reshape(NW, k, C)

    zerosd = jnp.zeros((rpt, d), jnp.float32)

    degp = _deg_kernel(cols3, nacc, k)
    d2 = degp.reshape(NW, nacc)[:, :n]

    y0 = _tc_scale0(d2, x)
    s1 = _agg_kernel(y0, rows3, cols3, zerosd, nacc, k, d)
    y1 = _tc_scale1(d2, s1, y0)
    s2 = _agg_kernel(y1, rows3, cols3, zerosd, nacc, k, d)
    return _tc_final(d2, s2, y1, W, b)


# speedup vs baseline: 33.7475x; 1.4340x over previous
import dataclasses
import functools

import jax
import jax.numpy as jnp
from jax import lax
from jax.experimental import pallas as pl
from jax.experimental.pallas import tpu as pltpu
from jax.experimental.pallas import tpu_sc as plsc

NC = 2
NS = 16
NW = NC * NS
C = 128
DUMMY = 16


def _mesh():
    return plsc.VectorSubcoreMesh(
        core_axis_name="c", subcore_axis_name="s", num_cores=NC, num_subcores=NS
    )


def _deg_kernel(cols3, nacc, k):
    @functools.partial(
        pl.kernel,
        out_type=jax.ShapeDtypeStruct((NW * nacc,), jnp.float32),
        mesh=_mesh(),
        scratch_types=[
            pltpu.VMEM((k, C), jnp.int32),
            pltpu.VMEM((nacc,), jnp.float32),
        ],
        compiler_params=dataclasses.replace(
            pltpu.CompilerParams(), needs_layout_passes=False
        ),
    )
    def kern(cols_hbm, out_hbm, colv, hist):
        c = lax.axis_index("c")
        s = lax.axis_index("s")
        wid = s * NC + c
        pltpu.sync_copy(cols_hbm.at[wid], colv)

        @pl.loop(0, nacc // 16)
        def _(i):
            hist[pl.ds(i * 16, 16)] = jnp.zeros((16,), jnp.float32)

        ones = jnp.ones((16,), jnp.float32)

        @pl.loop(0, k)
        def _(kk):
            @pl.loop(0, C // 16)
            def _(j):
                idx = colv[kk, pl.ds(j * 16, 16)]
                plsc.addupdate_scatter(hist, [idx], ones)

        pltpu.sync_copy(hist, out_hbm.at[pl.ds(wid * nacc, nacc)])

    return kern(cols3)


def _agg_kernel(y, rows3, cols3, zeros, nacc, k, d):
    rpt = nacc // NS
    k2 = k // 2

    @functools.partial(
        pl.kernel,
        out_type=jax.ShapeDtypeStruct((NC, nacc, d), jnp.float32),
        mesh=_mesh(),
        scratch_types=[
            pltpu.VMEM((k2, C), jnp.int32),
            pltpu.VMEM((k2, C), jnp.int32),
            pltpu.VMEM((C, d), jnp.float32),
            pltpu.VMEM((C, d), jnp.float32),
            pltpu.VMEM_SHARED((nacc, d), jnp.float32),
            pltpu.SemaphoreType.DMA,
            pltpu.SemaphoreType.DMA,
        ],
    )
    def kern(y_hbm, rows_hbm, cols_hbm, zeros_hbm, out_hbm,
             rowv, colv, buf0, buf1, acc, sem0, sem1):
        c = lax.axis_index("c")
        s = lax.axis_index("s")
        wid = s * NC + c
        pltpu.sync_copy(zeros_hbm, acc.at[pl.ds(s * rpt, rpt)])
        plsc.subcore_barrier()

        for h in range(2):
            pltpu.sync_copy(rows_hbm.at[wid, pl.ds(h * k2, k2)], rowv)
            pltpu.sync_copy(cols_hbm.at[wid, pl.ds(h * k2, k2)], colv)

            pltpu.async_copy(y_hbm.at[rowv.at[0]], buf0, sem0)

            @pl.loop(0, k2 // 2)
            def _(g):
                b = g * 2
                pltpu.async_copy(y_hbm.at[rowv.at[b + 1]], buf1, sem1)
                pltpu.make_async_copy(y_hbm.at[rowv.at[b]], buf0, sem0).wait()
                pltpu.sync_copy(buf0, acc.at[colv.at[b]], add=True)
                nxt = jnp.minimum(b + 2, k2 - 2)
                pltpu.async_copy(y_hbm.at[rowv.at[nxt]], buf0, sem0)
                pltpu.make_async_copy(y_hbm.at[rowv.at[b + 1]], buf1, sem1).wait()
                pltpu.sync_copy(buf1, acc.at[colv.at[b + 1]], add=True)

            pltpu.make_async_copy(y_hbm.at[rowv.at[k2 - 2]], buf0, sem0).wait()

        plsc.subcore_barrier()
        pltpu.sync_copy(
            acc.at[pl.ds(s * rpt, rpt)], out_hbm.at[c, pl.ds(s * rpt, rpt)]
        )

    return kern(y, rows3, cols3, zeros)


def _tc_scale0(d2, x):
    n, d = x.shape

    def body(d_ref, x_ref, y_ref):
        deg = jnp.sum(d_ref[...], axis=0) + 1.0
        dis = lax.rsqrt(deg)
        y_ref[...] = x_ref[...] * dis[:, None]

    return pl.pallas_call(
        body, out_shape=jax.ShapeDtypeStruct((n, d), jnp.float32)
    )(d2, x)


def _tc_scale1(d2, sp, y0):
    n, d = y0.shape

    def body(d_ref, s_ref, y0_ref, out_ref):
        deg = jnp.sum(d_ref[...], axis=0) + 1.0
        t = s_ref[0, pl.ds(0, n), :] + s_ref[1, pl.ds(0, n), :] + y0_ref[...]
        out_ref[...] = t * (1.0 / deg)[:, None]

    return pl.pallas_call(
        body, out_shape=jax.ShapeDtypeStruct((n, d), jnp.float32)
    )(d2, sp, y0)


def _tc_final(d2, sp, y1, W, b):
    n, d = y1.shape

    def body(d_ref, s_ref, y1_ref, w_ref, b_ref, out_ref):
        deg = jnp.sum(d_ref[...], axis=0) + 1.0
        dis = lax.rsqrt(deg)
        t = s_ref[0, pl.ds(0, n), :] + s_ref[1, pl.ds(0, n), :] + y1_ref[...]
        t = t * dis[:, None]
        out_ref[...] = (
            lax.dot_general(
                t, w_ref[...], (((1,), (1,)), ((), ())),
                preferred_element_type=jnp.float32,
            )
            + b_ref[...][None, :]
        )

    return pl.pallas_call(
        body, out_shape=jax.ShapeDtypeStruct((n, W.shape[0]), jnp.float32)
    )(d2, sp, y1, W, b)


def kernel(x, edge_index, W, b):
    n, d = x.shape
    e = edge_index.shape[1]

    k = -(-e // (NW * C))
    k = -(-k // 4) * 4
    e_pad = NW * C * k
    pad = e_pad - e
    nacc = -(-(n + DUMMY) // (NS * 8)) * (NS * 8)
    rpt = nacc // NS

    row = edge_index[0].astype(jnp.int32)
    col = edge_index[1].astype(jnp.int32)
    ar = jnp.arange(pad, dtype=jnp.int32)
    rowp = jnp.concatenate([row, (ar * 37) % n])
    colp = jnp.concatenate([col, n + (ar % DUMMY)])
    rows3 = rowp.reshape(NW, k, C)
    cols3 = colp.reshape(NW, k, C)

    zerosd = jnp.zeros((rpt, d), jnp.float32)

    degp = _deg_kernel(cols3, nacc, k)
    d2 = degp.reshape(NW, nacc)[:, :n]

    y0 = _tc_scale0(d2, x)
    s1 = _agg_kernel(y0, rows3, cols3, zerosd, nacc, k, d)
    y1 = _tc_scale1(d2, s1, y0)
    s2 = _agg_kernel(y1, rows3, cols3, zerosd, nacc, k, d)
    return _tc_final(d2, s2, y1, W, b)


# speedup vs baseline: 34.1258x; 1.0112x over previous
import dataclasses
import functools

import jax
import jax.numpy as jnp
from jax import lax
from jax.experimental import pallas as pl
from jax.experimental.pallas import tpu as pltpu
from jax.experimental.pallas import tpu_sc as plsc

NC = 2
NS = 16
NW = NC * NS
C = 64
DUMMY = 16


def _mesh():
    return plsc.VectorSubcoreMesh(
        core_axis_name="c", subcore_axis_name="s", num_cores=NC, num_subcores=NS
    )


def _deg_kernel(cols3, nacc, k):
    @functools.partial(
        pl.kernel,
        out_type=jax.ShapeDtypeStruct((NW * nacc,), jnp.float32),
        mesh=_mesh(),
        scratch_types=[
            pltpu.VMEM((k, C), jnp.int32),
            pltpu.VMEM((nacc,), jnp.float32),
        ],
        compiler_params=dataclasses.replace(
            pltpu.CompilerParams(), needs_layout_passes=False
        ),
    )
    def kern(cols_hbm, out_hbm, colv, hist):
        c = lax.axis_index("c")
        s = lax.axis_index("s")
        wid = s * NC + c
        pltpu.sync_copy(cols_hbm.at[wid], colv)

        @pl.loop(0, nacc // 16)
        def _(i):
            hist[pl.ds(i * 16, 16)] = jnp.zeros((16,), jnp.float32)

        ones = jnp.ones((16,), jnp.float32)

        @pl.loop(0, k)
        def _(kk):
            @pl.loop(0, C // 16)
            def _(j):
                idx = colv[kk, pl.ds(j * 16, 16)]
                plsc.addupdate_scatter(hist, [idx], ones)

        pltpu.sync_copy(hist, out_hbm.at[pl.ds(wid * nacc, nacc)])

    return kern(cols3)


def _agg_kernel(y, rows3, cols3, zeros, nacc, k, d):
    rpt = nacc // NS
    qk = k // 4
    NB = 4

    @functools.partial(
        pl.kernel,
        out_type=jax.ShapeDtypeStruct((NC, nacc, d), jnp.float32),
        mesh=_mesh(),
        scratch_types=[
            pltpu.VMEM((qk, C), jnp.int32),
            pltpu.VMEM((qk, C), jnp.int32),
            [pltpu.VMEM((C, d), jnp.float32)] * NB,
            pltpu.VMEM_SHARED((nacc, d), jnp.float32),
            [pltpu.SemaphoreType.DMA] * NB,
        ],
    )
    def kern(y_hbm, rows_hbm, cols_hbm, zeros_hbm, out_hbm,
             rowv, colv, bufs, acc, sems):
        c = lax.axis_index("c")
        s = lax.axis_index("s")
        wid = s * NC + c
        pltpu.sync_copy(zeros_hbm, acc.at[pl.ds(s * rpt, rpt)])
        plsc.subcore_barrier()

        for q in range(4):
            pltpu.sync_copy(rows_hbm.at[wid, pl.ds(q * qk, qk)], rowv)
            pltpu.sync_copy(cols_hbm.at[wid, pl.ds(q * qk, qk)], colv)

            for j in range(NB - 1):
                pltpu.async_copy(y_hbm.at[rowv.at[j]], bufs[j], sems[j])

            @pl.loop(0, qk // NB)
            def _(g):
                for j in range(NB):
                    b = g * NB + j
                    m = (j + NB - 1) % NB
                    bn = jnp.minimum(b + NB - 1, qk - 1)
                    pltpu.async_copy(y_hbm.at[rowv.at[bn]], bufs[m], sems[m])
                    pltpu.make_async_copy(
                        y_hbm.at[rowv.at[b]], bufs[j], sems[j]
                    ).wait()
                    pltpu.sync_copy(bufs[j], acc.at[colv.at[b]], add=True)

            for m in range(NB - 1):
                pltpu.make_async_copy(
                    y_hbm.at[rowv.at[qk - 1]], bufs[m], sems[m]
                ).wait()

        plsc.subcore_barrier()
        pltpu.sync_copy(
            acc.at[pl.ds(s * rpt, rpt)], out_hbm.at[c, pl.ds(s * rpt, rpt)]
        )

    return kern(y, rows3, cols3, zeros)


def _tc_scale0(d2, x):
    n, d = x.shape

    def body(d_ref, x_ref, y_ref):
        deg = jnp.sum(d_ref[...], axis=0) + 1.0
        dis = lax.rsqrt(deg)
        y_ref[...] = x_ref[...] * dis[:, None]

    return pl.pallas_call(
        body, out_shape=jax.ShapeDtypeStruct((n, d), jnp.float32)
    )(d2, x)


def _tc_scale1(d2, sp, y0):
    n, d = y0.shape

    def body(d_ref, s_ref, y0_ref, out_ref):
        deg = jnp.sum(d_ref[...], axis=0) + 1.0
        t = s_ref[0, pl.ds(0, n), :] + s_ref[1, pl.ds(0, n), :] + y0_ref[...]
        out_ref[...] = t * (1.0 / deg)[:, None]

    return pl.pallas_call(
        body, out_shape=jax.ShapeDtypeStruct((n, d), jnp.float32)
    )(d2, sp, y0)


def _tc_final(d2, sp, y1, W, b):
    n, d = y1.shape

    def body(d_ref, s_ref, y1_ref, w_ref, b_ref, out_ref):
        deg = jnp.sum(d_ref[...], axis=0) + 1.0
        dis = lax.rsqrt(deg)
        t = s_ref[0, pl.ds(0, n), :] + s_ref[1, pl.ds(0, n), :] + y1_ref[...]
        t = t * dis[:, None]
        out_ref[...] = (
            lax.dot_general(
                t, w_ref[...], (((1,), (1,)), ((), ())),
                preferred_element_type=jnp.float32,
            )
            + b_ref[...][None, :]
        )

    return pl.pallas_call(
        body, out_shape=jax.ShapeDtypeStruct((n, W.shape[0]), jnp.float32)
    )(d2, sp, y1, W, b)


def kernel(x, edge_index, W, b):
    n, d = x.shape
    e = edge_index.shape[1]

    k = -(-e // (NW * C))
    k = -(-k // 16) * 16
    e_pad = NW * C * k
    pad = e_pad - e
    nacc = -(-(n + DUMMY) // (NS * 8)) * (NS * 8)
    rpt = nacc // NS

    row = edge_index[0].astype(jnp.int32)
    col = edge_index[1].astype(jnp.int32)
    ar = jnp.arange(pad, dtype=jnp.int32)
    rowp = jnp.concatenate([row, (ar * 37) % n])
    colp = jnp.concatenate([col, n + (ar % DUMMY)])
    rows3 = rowp.reshape(NW, k, C)
    cols3 = colp.reshape(NW, k, C)

    zerosd = jnp.zeros((rpt, d), jnp.float32)

    degp = _deg_kernel(cols3, nacc, k)
    d2 = degp.reshape(NW, nacc)[:, :n]

    y0 = _tc_scale0(d2, x)
    s1 = _agg_kernel(y0, rows3, cols3, zerosd, nacc, k, d)
    y1 = _tc_scale1(d2, s1, y0)
    s2 = _agg_kernel(y1, rows3, cols3, zerosd, nacc, k, d)
    return _tc_final(d2, s2, y1, W, b)


# speedup vs baseline: 34.3783x; 1.0074x over previous
import dataclasses
import functools

import jax
import jax.numpy as jnp
from jax import lax
from jax.experimental import pallas as pl
from jax.experimental.pallas import tpu as pltpu
from jax.experimental.pallas import tpu_sc as plsc

NC = 2
NS = 16
NW = NC * NS
C = 64
NB = 4
DUMMY = 16


def _mesh():
    return plsc.VectorSubcoreMesh(
        core_axis_name="c", subcore_axis_name="s", num_cores=NC, num_subcores=NS
    )


def _deg_kernel(cols3, nacc, k):
    @functools.partial(
        pl.kernel,
        out_type=jax.ShapeDtypeStruct((NW * nacc,), jnp.float32),
        mesh=_mesh(),
        scratch_types=[
            pltpu.VMEM((k, C), jnp.int32),
            pltpu.VMEM((nacc,), jnp.float32),
        ],
        compiler_params=dataclasses.replace(
            pltpu.CompilerParams(), needs_layout_passes=False
        ),
    )
    def kern(cols_hbm, out_hbm, colv, hist):
        c = lax.axis_index("c")
        s = lax.axis_index("s")
        wid = s * NC + c
        pltpu.sync_copy(cols_hbm.at[wid], colv)

        @pl.loop(0, nacc // 64)
        def _(i):
            for u in range(4):
                hist[pl.ds(i * 64 + u * 16, 16)] = jnp.zeros((16,), jnp.float32)

        ones = jnp.ones((16,), jnp.float32)

        @pl.loop(0, k)
        def _(kk):
            for j in range(C // 16):
                idx = colv[kk, pl.ds(j * 16, 16)]
                plsc.addupdate_scatter(hist, [idx], ones)

        pltpu.sync_copy(hist, out_hbm.at[pl.ds(wid * nacc, nacc)])

    return kern(cols3)


def _agg_kernel(y, rows3, cols3, zeros, nacc, k, d):
    rpt = nacc // NS
    qk = k // 4

    @functools.partial(
        pl.kernel,
        out_type=jax.ShapeDtypeStruct((NC, nacc, d), jnp.float32),
        mesh=_mesh(),
        scratch_types=[
            pltpu.VMEM((qk, C), jnp.int32),
            pltpu.VMEM((qk, C), jnp.int32),
            [pltpu.VMEM((C, d), jnp.float32)] * NB,
            pltpu.VMEM_SHARED((nacc, d), jnp.float32),
            [pltpu.SemaphoreType.DMA] * NB,
        ],
    )
    def kern(y_hbm, rows_hbm, cols_hbm, zeros_hbm, out_hbm,
             rowv, colv, bufs, acc, sems):
        c = lax.axis_index("c")
        s = lax.axis_index("s")
        wid = s * NC + c
        pltpu.sync_copy(zeros_hbm, acc.at[pl.ds(s * rpt, rpt)])
        plsc.subcore_barrier()

        for q in range(4):
            pltpu.sync_copy(rows_hbm.at[wid, pl.ds(q * qk, qk)], rowv)
            pltpu.sync_copy(cols_hbm.at[wid, pl.ds(q * qk, qk)], colv)

            for j in range(NB - 1):
                pltpu.async_copy(y_hbm.at[rowv.at[j]], bufs[j], sems[j])

            @pl.loop(0, qk // NB)
            def _(g):
                for j in range(NB):
                    b = g * NB + j
                    m = (j + NB - 1) % NB
                    bn = jnp.minimum(b + NB - 1, qk - 1)
                    pltpu.async_copy(y_hbm.at[rowv.at[bn]], bufs[m], sems[m])
                    pltpu.make_async_copy(
                        y_hbm.at[rowv.at[b]], bufs[j], sems[j]
                    ).wait()
                    pltpu.sync_copy(bufs[j], acc.at[colv.at[b]], add=True)

            for m in range(NB - 1):
                pltpu.make_async_copy(
                    y_hbm.at[rowv.at[qk - 1]], bufs[m], sems[m]
                ).wait()

        plsc.subcore_barrier()
        pltpu.sync_copy(
            acc.at[pl.ds(s * rpt, rpt)], out_hbm.at[c, pl.ds(s * rpt, rpt)]
        )

    return kern(y, rows3, cols3, zeros)


def _tc_scale0(d2, x):
    n, d = x.shape

    def body(d_ref, x_ref, y_ref):
        deg = jnp.sum(d_ref[...], axis=0) + 1.0
        dis = lax.rsqrt(deg)
        y_ref[...] = x_ref[...] * dis[:, None]

    return pl.pallas_call(
        body, out_shape=jax.ShapeDtypeStruct((n, d), jnp.float32)
    )(d2, x)


def _tc_scale1(d2, sp, y0):
    n, d = y0.shape

    def body(d_ref, s_ref, y0_ref, out_ref):
        deg = jnp.sum(d_ref[...], axis=0) + 1.0
        t = s_ref[0, pl.ds(0, n), :] + s_ref[1, pl.ds(0, n), :] + y0_ref[...]
        out_ref[...] = t * (1.0 / deg)[:, None]

    return pl.pallas_call(
        body, out_shape=jax.ShapeDtypeStruct((n, d), jnp.float32)
    )(d2, sp, y0)


def _tc_final(d2, sp, y1, W, b):
    n, d = y1.shape

    def body(d_ref, s_ref, y1_ref, w_ref, b_ref, out_ref):
        deg = jnp.sum(d_ref[...], axis=0) + 1.0
        dis = lax.rsqrt(deg)
        t = s_ref[0, pl.ds(0, n), :] + s_ref[1, pl.ds(0, n), :] + y1_ref[...]
        t = t * dis[:, None]
        out_ref[...] = (
            lax.dot_general(
                t, w_ref[...], (((1,), (1,)), ((), ())),
                preferred_element_type=jnp.float32,
            )
            + b_ref[...][None, :]
        )

    return pl.pallas_call(
        body, out_shape=jax.ShapeDtypeStruct((n, W.shape[0]), jnp.float32)
    )(d2, sp, y1, W, b)


def kernel(x, edge_index, W, b):
    n, d = x.shape
    e = edge_index.shape[1]

    k = -(-e // (NW * C))
    k = -(-k // (4 * NB)) * (4 * NB)
    e_pad = NW * C * k
    pad = e_pad - e
    nacc = -(-(n + DUMMY) // (NS * 8)) * (NS * 8)
    rpt = nacc // NS

    row = edge_index[0].astype(jnp.int32)
    col = edge_index[1].astype(jnp.int32)
    ar = jnp.arange(pad, dtype=jnp.int32)
    rowp = jnp.concatenate([row, (ar * 37) % n])
    colp = jnp.concatenate([col, n + (ar % DUMMY)])
    rows3 = rowp.reshape(NW, k, C)
    cols3 = colp.reshape(NW, k, C)

    zerosd = jnp.zeros((rpt, d), jnp.float32)

    degp = _deg_kernel(cols3, nacc, k)
    d2 = degp.reshape(NW, nacc)[:, :n]

    y0 = _tc_scale0(d2, x)
    s1 = _agg_kernel(y0, rows3, cols3, zerosd, nacc, k, d)
    y1 = _tc_scale1(d2, s1, y0)
    s2 = _agg_kernel(y1, rows3, cols3, zerosd, nacc, k, d)
    return _tc_final(d2, s2, y1, W, b)
